# baseline (speedup 1.0000x reference)
import functools
import math

import jax
import jax.numpy as jnp
from jax import lax
from jax.experimental import pallas as pl
from jax.experimental.pallas import tpu as pltpu
from jax.experimental.pallas import tpu_sc as plsc

D = 64
SCALE = math.sqrt(D)
L = 16
NC, NS = 2, 16
NW = NC * NS

RB = 16


def _emb_body(n_chunks, rows_per_w, h, x_hbm, table_hbm, out_hbm,
              idx_v, rows_v, sem):
    wid = lax.axis_index("s") * NC + lax.axis_index("c")
    base = wid * rows_per_w

    def chunk_body(g, carry):
        r0 = pl.multiple_of(base + g * RB, 8)
        pltpu.sync_copy(x_hbm.at[pl.ds(r0, RB)], idx_v)
        cps = [
            pltpu.async_copy(table_hbm.at[idx_v.at[j]], rows_v.at[j], sem)
            for j in range(RB)
        ]
        for cp in cps:
            cp.wait()

        def scale_row(r, c2):
            for c in range(D // L):
                sl = pl.ds(c * L, L)
                rows_v[r // h, r % h, sl] = rows_v[r // h, r % h, sl] * SCALE
            return c2

        lax.fori_loop(0, RB * h, scale_row, 0, unroll=2)

        pltpu.sync_copy(rows_v, out_hbm.at[pl.ds(r0, RB)])
        return carry

    lax.fori_loop(0, n_chunks, chunk_body, 0)


def kernel(x, table):
    b, h = x.shape
    assert b % (NW * RB) == 0
    rows_per_w = b // NW
    n_chunks = rows_per_w // RB

    mesh = plsc.VectorSubcoreMesh(core_axis_name="c", subcore_axis_name="s")
    emb = pl.kernel(
        functools.partial(_emb_body, n_chunks, rows_per_w, h),
        mesh=mesh,
        compiler_params=pltpu.CompilerParams(use_tc_tiling_on_sc=False),
        out_type=jax.ShapeDtypeStruct((b, h, D), jnp.float32),
        scratch_types=[
            pltpu.VMEM((RB, h), jnp.int32),
            pltpu.VMEM((RB, h, D), jnp.float32),
            pltpu.SemaphoreType.DMA,
        ],
    )
    return emb(x, table)
